# Initial kernel scaffold; baseline (speedup 1.0000x reference)
import jax
import jax.numpy as jnp
from jax.experimental import pallas as pl


def kernel(user_emb, item_emb, adj_values, adj_indices):
    raise NotImplementedError("write your pallas kernel here")


# speedup vs baseline: 3.1514x; 3.1514x over previous
import functools

import jax
import jax.numpy as jnp
from jax import lax
from jax.experimental import pallas as pl
from jax.experimental.pallas import tpu as pltpu
from jax.experimental.pallas import tpu_sc as plsc

USERS = 25000
NODES = 50000
QCOL = 16
NODESP = 50176
EDGES = 800000
LANES = 128
EROWS_PAD = 6400
RPT = EROWS_PAD // 16
CHUNK_R = 16
NCHUNK = RPT // CHUNK_R
NPT = NODESP // 16
ZROWS = 392
MCHUNK = 392


def _body(ego0, srcb, dstb, valb, o1, o2, o3, omean,
          sidx, didx, vbuf, rows, zbuf, mA, mB, acc, sem):
    c = lax.axis_index("c")
    s = lax.axis_index("s")
    rbase = s * NPT
    edge0 = s * RPT

    z16 = jnp.zeros((16,), jnp.float32)

    @pl.loop(0, ZROWS)
    def _(i):
        zbuf[i, :] = z16

    def layer(prev, cur):
        @pl.loop(0, 2)
        def _(p):
            q = 2 * c + p
            for j in range(NPT // ZROWS):
                pltpu.sync_copy(zbuf, acc.at[pl.ds(rbase + j * ZROWS, ZROWS)])
            plsc.subcore_barrier()

            @pl.loop(0, NCHUNK)
            def _(k):
                er = q * EROWS_PAD + edge0 + k * CHUNK_R
                ed = edge0 + k * CHUNK_R
                pltpu.sync_copy(srcb.at[pl.ds(er, CHUNK_R)], sidx)
                pltpu.sync_copy(dstb.at[pl.ds(ed, CHUNK_R)], didx)
                pltpu.sync_copy(valb.at[pl.ds(ed, CHUNK_R)], vbuf)
                for j in range(CHUNK_R):
                    pltpu.async_copy(prev.at[sidx.at[j]], rows.at[j], sem).wait()

                @pl.loop(0, CHUNK_R * (LANES // 16))
                def _(g):
                    j = g // (LANES // 16)
                    col = (g % (LANES // 16)) * 16
                    v16 = vbuf[j, pl.ds(col, 16)]
                    for lane in range(16):
                        b = jnp.broadcast_to(v16[lane], (16,))
                        rows[j, col + lane, :] = rows[j, col + lane, :] * b

                for j in range(CHUNK_R):
                    pltpu.sync_copy(rows.at[j], acc.at[didx.at[j]], add=True)

            plsc.subcore_barrier()
            pltpu.sync_copy(acc.at[pl.ds(rbase, NPT)],
                            cur.at[pl.ds(q * NODESP + rbase, NPT)])
            plsc.subcore_barrier()

    layer(ego0, o1)
    layer(o1, o2)
    layer(o2, o3)

    quarter = jnp.float32(0.25)

    @pl.loop(0, 2)
    def _(p):
        q = 2 * c + p
        for w in range(NPT // MCHUNK):
            m0 = q * NODESP + rbase + w * MCHUNK
            pltpu.sync_copy(ego0.at[pl.ds(m0, MCHUNK)], mA)
            for o in (o1, o2, o3):
                pltpu.sync_copy(o.at[pl.ds(m0, MCHUNK)], mB)

                @pl.loop(0, MCHUNK)
                def _(i):
                    mA[i, :] = mA[i, :] + mB[i, :]

            @pl.loop(0, MCHUNK)
            def _(i):
                mA[i, :] = mA[i, :] * quarter

            pltpu.sync_copy(mA, omean.at[pl.ds(m0, MCHUNK)])


def _make_sc_call():
    mesh = plsc.VectorSubcoreMesh(core_axis_name="c", subcore_axis_name="s")
    f32 = jnp.float32
    return functools.partial(
        pl.kernel,
        mesh=mesh,
        compiler_params=pltpu.CompilerParams(use_tc_tiling_on_sc=False),
        out_type=[
            jax.ShapeDtypeStruct((4 * NODESP, QCOL), f32),
            jax.ShapeDtypeStruct((4 * NODESP, QCOL), f32),
            jax.ShapeDtypeStruct((4 * NODESP, QCOL), f32),
            jax.ShapeDtypeStruct((4 * NODESP, QCOL), f32),
        ],
        scratch_types=[
            pltpu.VMEM((CHUNK_R, LANES), jnp.int32),
            pltpu.VMEM((CHUNK_R, LANES), jnp.int32),
            pltpu.VMEM((CHUNK_R, LANES), f32),
            pltpu.VMEM((CHUNK_R, LANES, QCOL), f32),
            pltpu.VMEM((ZROWS, QCOL), f32),
            pltpu.VMEM((MCHUNK, QCOL), f32),
            pltpu.VMEM((MCHUNK, QCOL), f32),
            pltpu.VMEM_SHARED((NODESP, QCOL), f32),
            pltpu.SemaphoreType.DMA,
        ],
    )(_body)


def kernel(user_emb, item_emb, adj_values, adj_indices):
    ego0 = jnp.concatenate([user_emb, item_emb], axis=0)
    zrows = jnp.zeros((NODESP - NODES, QCOL), jnp.float32)
    ego_q = jnp.concatenate(
        [x for i in range(4) for x in (ego0[:, i * QCOL:(i + 1) * QCOL], zrows)],
        axis=0)

    src = adj_indices[0]
    dst = adj_indices[1]
    pad = EROWS_PAD * LANES - EDGES
    srcp = jnp.concatenate([src, jnp.zeros((pad,), jnp.int32)])
    srcb = jnp.concatenate(
        [srcp + i * NODESP for i in range(4)]).reshape(4 * EROWS_PAD, LANES)
    dstb = jnp.concatenate([dst, jnp.zeros((pad,), jnp.int32)])
    dstb = dstb.reshape(EROWS_PAD, LANES)
    valb = jnp.concatenate([adj_values, jnp.zeros((pad,), jnp.float32)])
    valb = valb.reshape(EROWS_PAD, LANES)

    _, _, _, mean = _make_sc_call()(ego_q, srcb, dstb, valb)
    avg = jnp.concatenate(
        [mean[i * NODESP:i * NODESP + NODES] for i in range(4)], axis=1)
    return avg[:USERS], avg[USERS:]


# speedup vs baseline: 5.7056x; 1.8105x over previous
import functools

import jax
import jax.numpy as jnp
from jax import lax
from jax.experimental import pallas as pl
from jax.experimental.pallas import tpu as pltpu
from jax.experimental.pallas import tpu_sc as plsc

USERS = 25000
NODES = 50000
QCOL = 16
NODESP = 50176
EDGES = 800000
LANES = 128
EROWS_PAD = 6400
RPT = EROWS_PAD // 16
CHUNK_R = 8
NCHUNK = RPT // CHUNK_R
NPT = NODESP // 16
ZROWS = 392
MCHUNK = 392


def _body(ego0, srcb, dstb, valb, o1, o2, o3, omean,
          sidx0, didx0, vbuf0, rows0, sidx1, didx1, vbuf1, rows1,
          zbuf, mA, mB, acc, gsem0, gsem1, ssem):
    c = lax.axis_index("c")
    s = lax.axis_index("s")
    rbase = s * NPT
    edge0 = s * RPT

    bufs = ((sidx0, didx0, vbuf0, rows0, gsem0),
            (sidx1, didx1, vbuf1, rows1, gsem1))

    z16 = jnp.zeros((16,), jnp.float32)

    @pl.loop(0, ZROWS)
    def _(i):
        zbuf[i, :] = z16

    def layer(prev, cur):
        @pl.loop(0, 2)
        def _(p):
            q = 2 * c + p
            for j in range(NPT // ZROWS):
                pltpu.sync_copy(zbuf, acc.at[pl.ds(rbase + j * ZROWS, ZROWS)])
            plsc.subcore_barrier()

            def load_and_fire(k, b):
                si, di, vb, ro, gs = bufs[b]
                pltpu.sync_copy(
                    srcb.at[pl.ds(q * EROWS_PAD + edge0 + k * CHUNK_R, CHUNK_R)],
                    si)
                pltpu.sync_copy(dstb.at[pl.ds(edge0 + k * CHUNK_R, CHUNK_R)], di)
                pltpu.sync_copy(valb.at[pl.ds(edge0 + k * CHUNK_R, CHUNK_R)], vb)
                for j in range(CHUNK_R):
                    pltpu.async_copy(prev.at[si.at[j]], ro.at[j], gs)

            def process(b):
                si, di, vb, ro, gs = bufs[b]
                for j in range(CHUNK_R):
                    pltpu.make_async_copy(prev.at[si.at[j]], ro.at[j], gs).wait()

                @pl.loop(0, CHUNK_R * (LANES // 16))
                def _(g):
                    j = g // (LANES // 16)
                    col = (g % (LANES // 16)) * 16
                    v16 = vb[j, pl.ds(col, 16)]
                    for lane in range(16):
                        bc = jnp.broadcast_to(v16[lane], (16,))
                        ro[j, col + lane, :] = ro[j, col + lane, :] * bc

                scat = [pltpu.async_copy(ro.at[j], acc.at[di.at[j]], ssem,
                                         add=True)
                        for j in range(CHUNK_R)]
                for d in scat:
                    d.wait()

            load_and_fire(0, 0)

            @pl.loop(0, NCHUNK, step=2)
            def _(k):
                load_and_fire(k + 1, 1)
                process(0)

                @pl.when(k + 2 < NCHUNK)
                def _():
                    load_and_fire(k + 2, 0)

                process(1)

            plsc.subcore_barrier()
            pltpu.sync_copy(acc.at[pl.ds(rbase, NPT)],
                            cur.at[pl.ds(q * NODESP + rbase, NPT)])
            plsc.subcore_barrier()

    layer(ego0, o1)
    layer(o1, o2)
    layer(o2, o3)

    quarter = jnp.float32(0.25)

    @pl.loop(0, 2)
    def _(p):
        q = 2 * c + p
        for w in range(NPT // MCHUNK):
            m0 = q * NODESP + rbase + w * MCHUNK
            pltpu.sync_copy(ego0.at[pl.ds(m0, MCHUNK)], mA)
            for o in (o1, o2, o3):
                pltpu.sync_copy(o.at[pl.ds(m0, MCHUNK)], mB)

                @pl.loop(0, MCHUNK)
                def _(i):
                    mA[i, :] = mA[i, :] + mB[i, :]

            @pl.loop(0, MCHUNK)
            def _(i):
                mA[i, :] = mA[i, :] * quarter

            pltpu.sync_copy(mA, omean.at[pl.ds(m0, MCHUNK)])


def _make_sc_call():
    mesh = plsc.VectorSubcoreMesh(core_axis_name="c", subcore_axis_name="s")
    f32 = jnp.float32
    return functools.partial(
        pl.kernel,
        mesh=mesh,
        compiler_params=pltpu.CompilerParams(use_tc_tiling_on_sc=False),
        out_type=[
            jax.ShapeDtypeStruct((4 * NODESP, QCOL), f32),
            jax.ShapeDtypeStruct((4 * NODESP, QCOL), f32),
            jax.ShapeDtypeStruct((4 * NODESP, QCOL), f32),
            jax.ShapeDtypeStruct((4 * NODESP, QCOL), f32),
        ],
        scratch_types=[
            pltpu.VMEM((CHUNK_R, LANES), jnp.int32),
            pltpu.VMEM((CHUNK_R, LANES), jnp.int32),
            pltpu.VMEM((CHUNK_R, LANES), f32),
            pltpu.VMEM((CHUNK_R, LANES, QCOL), f32),
            pltpu.VMEM((CHUNK_R, LANES), jnp.int32),
            pltpu.VMEM((CHUNK_R, LANES), jnp.int32),
            pltpu.VMEM((CHUNK_R, LANES), f32),
            pltpu.VMEM((CHUNK_R, LANES, QCOL), f32),
            pltpu.VMEM((ZROWS, QCOL), f32),
            pltpu.VMEM((MCHUNK, QCOL), f32),
            pltpu.VMEM((MCHUNK, QCOL), f32),
            pltpu.VMEM_SHARED((NODESP, QCOL), f32),
            pltpu.SemaphoreType.DMA,
            pltpu.SemaphoreType.DMA,
            pltpu.SemaphoreType.DMA,
        ],
    )(_body)


def kernel(user_emb, item_emb, adj_values, adj_indices):
    ego0 = jnp.concatenate([user_emb, item_emb], axis=0)
    zrows = jnp.zeros((NODESP - NODES, QCOL), jnp.float32)
    ego_q = jnp.concatenate(
        [x for i in range(4) for x in (ego0[:, i * QCOL:(i + 1) * QCOL], zrows)],
        axis=0)

    src = adj_indices[0]
    dst = adj_indices[1]
    pad = EROWS_PAD * LANES - EDGES
    srcp = jnp.concatenate([src, jnp.zeros((pad,), jnp.int32)])
    srcb = jnp.concatenate(
        [srcp + i * NODESP for i in range(4)]).reshape(4 * EROWS_PAD, LANES)
    dstb = jnp.concatenate([dst, jnp.zeros((pad,), jnp.int32)])
    dstb = dstb.reshape(EROWS_PAD, LANES)
    valb = jnp.concatenate([adj_values, jnp.zeros((pad,), jnp.float32)])
    valb = valb.reshape(EROWS_PAD, LANES)

    _, _, _, mean = _make_sc_call()(ego_q, srcb, dstb, valb)
    avg = jnp.concatenate(
        [mean[i * NODESP:i * NODESP + NODES] for i in range(4)], axis=1)
    return avg[:USERS], avg[USERS:]
